# baseline (device time: 50459 ns/iter reference)
import os

import jax
import jax.numpy as jnp
from jax import lax
from jax.experimental import pallas as pl
from jax.experimental.pallas import tpu as pltpu

N_DEV = 16

_ABLATE = os.environ.get("ABLATE", "")
_NOAMAX = os.environ.get("NOAMAX", "") == "1"


def kernel(x, w_mat):
    m, k_per = x.shape
    k, n = w_mat.shape
    m_blk = m // N_DEV

    def body(x_ref, w_hbm_ref, out_ref, xb_ref, xg_ref, wstage_ref,
             amax_ref, send_sems, recv_sems, send2_sems, recv2_sems,
             wdma_sems):
        my_i = lax.axis_index("i")

        if _ABLATE != "compute":
            barrier_sem = pltpu.get_barrier_semaphore()
            for d in range(1, N_DEV):
                pl.semaphore_signal(
                    barrier_sem, inc=1,
                    device_id=((my_i + d) % N_DEV,),
                    device_id_type=pl.DeviceIdType.MESH,
                )
            pl.semaphore_wait(barrier_sem, N_DEV - 1)

        xb_ref[:, :] = x_ref[:, :].astype(jnp.bfloat16)

        a2a = []
        if _ABLATE != "compute":
            for d in range(1, N_DEV):
                t = (my_i + d) % N_DEV
                rdma = pltpu.make_async_remote_copy(
                    src_ref=xb_ref.at[pl.ds(t * m_blk, m_blk), :],
                    dst_ref=xg_ref.at[d],
                    send_sem=send_sems.at[d],
                    recv_sem=recv_sems.at[d],
                    device_id=(t,),
                    device_id_type=pl.DeviceIdType.MESH,
                )
                rdma.start()
                a2a.append(rdma)

        wdma = []
        for d in range(N_DEV):
            s = (my_i - d) % N_DEV
            cp = pltpu.make_async_copy(
                w_hbm_ref.at[pl.ds(s * k_per, k_per), :],
                wstage_ref.at[d],
                wdma_sems.at[d],
            )
            cp.start()
            wdma.append(cp)

        wdma[0].wait()
        out_ref[:, :] = jnp.dot(
            xb_ref[pl.ds(my_i * m_blk, m_blk), :],
            wstage_ref[0].astype(jnp.bfloat16),
            preferred_element_type=jnp.float32,
        )

        for d in range(1, N_DEV):
            if _ABLATE != "compute":
                a2a[d - 1].wait()
            wdma[d].wait()
            if _ABLATE != "comm":
                out_ref[:, :] = out_ref[:, :] + jnp.dot(
                    xg_ref[d],
                    wstage_ref[d].astype(jnp.bfloat16),
                    preferred_element_type=jnp.float32,
                )

        local_amax = jnp.max(jnp.abs(out_ref[:, :]))
        amax_ref[0, :] = jnp.full((128,), local_amax, jnp.float32)
        if _ABLATE != "compute" and not _NOAMAX:
            for r in range(4):
                p = lax.bitwise_xor(my_i, 1 << r)
                rdma = pltpu.make_async_remote_copy(
                    src_ref=amax_ref.at[0],
                    dst_ref=amax_ref.at[1 + r],
                    send_sem=send2_sems.at[r],
                    recv_sem=recv2_sems.at[r],
                    device_id=(p,),
                    device_id_type=pl.DeviceIdType.MESH,
                )
                rdma.start()
                rdma.wait()
                amax_ref[0, :] = jnp.maximum(
                    amax_ref[0, :], amax_ref[1 + r, :]
                )

        gmax = jnp.max(amax_ref[0, :])
        scale = gmax / 127.0
        q = jnp.clip(jnp.round(out_ref[:, :] / scale), -127.0, 127.0)
        out_ref[:, :] = q * scale

    return pl.pallas_call(
        body,
        out_shape=jax.ShapeDtypeStruct((m_blk, n), jnp.float32),
        in_specs=[
            pl.BlockSpec(memory_space=pltpu.VMEM),
            pl.BlockSpec(memory_space=pltpu.MemorySpace.HBM),
        ],
        out_specs=pl.BlockSpec(memory_space=pltpu.VMEM),
        scratch_shapes=[
            pltpu.VMEM((m, k_per), jnp.bfloat16),
            pltpu.VMEM((N_DEV, m_blk, k_per), jnp.bfloat16),
            pltpu.VMEM((N_DEV, k_per, n), jnp.float32),
            pltpu.VMEM((N_DEV, 128), jnp.float32),
            pltpu.SemaphoreType.DMA((N_DEV,)),
            pltpu.SemaphoreType.DMA((N_DEV,)),
            pltpu.SemaphoreType.DMA((N_DEV,)),
            pltpu.SemaphoreType.DMA((N_DEV,)),
            pltpu.SemaphoreType.DMA((N_DEV,)),
        ],
        compiler_params=pltpu.CompilerParams(
            collective_id=None if _ABLATE == "compute" else 0,
            vmem_limit_bytes=100 * 1024 * 1024,
        ),
    )(x, w_mat)


# device time: 47267 ns/iter; 1.0675x vs baseline; 1.0675x over previous
import os

import jax
import jax.numpy as jnp
from jax import lax
from jax.experimental import pallas as pl
from jax.experimental.pallas import tpu as pltpu

N_DEV = 16

_ABLATE = os.environ.get("ABLATE", "")
_NOAMAX = os.environ.get("NOAMAX", "") == "1"


def kernel(x, w_mat):
    m, k_per = x.shape
    k, n = w_mat.shape
    m_blk = m // N_DEV

    def body(x_ref, w_hbm_ref, out_ref, xb_ref, xg_ref, wstage_ref,
             amax_ref, send_sems, recv_sems, send2_sems, recv2_sems,
             wdma_sems):
        my_i = lax.axis_index("i")

        if _ABLATE != "compute":
            barrier_sem = pltpu.get_barrier_semaphore()
            for d in range(1, N_DEV):
                pl.semaphore_signal(
                    barrier_sem, inc=1,
                    device_id=((my_i + d) % N_DEV,),
                    device_id_type=pl.DeviceIdType.MESH,
                )
            pl.semaphore_wait(barrier_sem, N_DEV - 1)

        xb_ref[:, :] = x_ref[:, :].astype(jnp.bfloat16)

        a2a = []
        if _ABLATE != "compute":
            for d in range(1, N_DEV):
                t = (my_i + d) % N_DEV
                rdma = pltpu.make_async_remote_copy(
                    src_ref=xb_ref.at[pl.ds(t * m_blk, m_blk), :],
                    dst_ref=xg_ref.at[d],
                    send_sem=send_sems.at[d],
                    recv_sem=recv_sems.at[d],
                    device_id=(t,),
                    device_id_type=pl.DeviceIdType.MESH,
                )
                rdma.start()
                a2a.append(rdma)

        wdma = []
        for d in range(N_DEV):
            s = (my_i - d) % N_DEV
            cp = pltpu.make_async_copy(
                w_hbm_ref.at[pl.ds(s * k_per, k_per), :],
                wstage_ref.at[d],
                wdma_sems.at[d],
            )
            cp.start()
            wdma.append(cp)

        wdma[0].wait()
        out_ref[:, :] = jnp.dot(
            xb_ref[pl.ds(my_i * m_blk, m_blk), :],
            wstage_ref[0].astype(jnp.bfloat16),
            preferred_element_type=jnp.float32,
        )

        for d in range(1, N_DEV):
            if _ABLATE != "compute":
                a2a[d - 1].wait()
            wdma[d].wait()
            if _ABLATE != "comm":
                out_ref[:, :] = out_ref[:, :] + jnp.dot(
                    xg_ref[d],
                    wstage_ref[d].astype(jnp.bfloat16),
                    preferred_element_type=jnp.float32,
                )

        local_amax = jnp.max(jnp.abs(out_ref[:, :]))
        amax_ref[0, :] = jnp.full((128,), local_amax, jnp.float32)
        if _ABLATE != "compute" and not _NOAMAX:
            p4 = my_i % 4
            base = my_i - p4
            r1 = []
            for o in range(1, 4):
                t = base + (p4 + o) % 4
                rdma = pltpu.make_async_remote_copy(
                    src_ref=amax_ref.at[0],
                    dst_ref=amax_ref.at[o],
                    send_sem=send2_sems.at[o - 1],
                    recv_sem=recv2_sems.at[o - 1],
                    device_id=(t,),
                    device_id_type=pl.DeviceIdType.MESH,
                )
                rdma.start()
                r1.append(rdma)
            for rdma in r1:
                rdma.wait()
            amax_ref[0, :] = jnp.maximum(
                jnp.maximum(amax_ref[0, :], amax_ref[1, :]),
                jnp.maximum(amax_ref[2, :], amax_ref[3, :]),
            )
            zz = my_i // 4
            r2 = []
            for o in range(1, 4):
                t = ((zz + o) % 4) * 4 + p4
                rdma = pltpu.make_async_remote_copy(
                    src_ref=amax_ref.at[0],
                    dst_ref=amax_ref.at[4 + o],
                    send_sem=send2_sems.at[2 + o],
                    recv_sem=recv2_sems.at[2 + o],
                    device_id=(t,),
                    device_id_type=pl.DeviceIdType.MESH,
                )
                rdma.start()
                r2.append(rdma)
            for rdma in r2:
                rdma.wait()
            amax_ref[0, :] = jnp.maximum(
                jnp.maximum(amax_ref[0, :], amax_ref[5, :]),
                jnp.maximum(amax_ref[6, :], amax_ref[7, :]),
            )

        gmax = jnp.max(amax_ref[0, :])
        scale = gmax / 127.0
        q = jnp.clip(jnp.round(out_ref[:, :] / scale), -127.0, 127.0)
        out_ref[:, :] = q * scale

    return pl.pallas_call(
        body,
        out_shape=jax.ShapeDtypeStruct((m_blk, n), jnp.float32),
        in_specs=[
            pl.BlockSpec(memory_space=pltpu.VMEM),
            pl.BlockSpec(memory_space=pltpu.MemorySpace.HBM),
        ],
        out_specs=pl.BlockSpec(memory_space=pltpu.VMEM),
        scratch_shapes=[
            pltpu.VMEM((m, k_per), jnp.bfloat16),
            pltpu.VMEM((N_DEV, m_blk, k_per), jnp.bfloat16),
            pltpu.VMEM((N_DEV, k_per, n), jnp.float32),
            pltpu.VMEM((N_DEV, 128), jnp.float32),
            pltpu.SemaphoreType.DMA((N_DEV,)),
            pltpu.SemaphoreType.DMA((N_DEV,)),
            pltpu.SemaphoreType.DMA((N_DEV,)),
            pltpu.SemaphoreType.DMA((N_DEV,)),
            pltpu.SemaphoreType.DMA((N_DEV,)),
        ],
        compiler_params=pltpu.CompilerParams(
            collective_id=None if _ABLATE == "compute" else 0,
            vmem_limit_bytes=100 * 1024 * 1024,
        ),
    )(x, w_mat)


# device time: 46156 ns/iter; 1.0932x vs baseline; 1.0241x over previous
import os

import jax
import jax.numpy as jnp
from jax import lax
from jax.experimental import pallas as pl
from jax.experimental.pallas import tpu as pltpu

N_DEV = 16

_ABLATE = os.environ.get("ABLATE", "")
_NOAMAX = os.environ.get("NOAMAX", "") == "1"


def kernel(x, w_mat):
    m, k_per = x.shape
    k, n = w_mat.shape
    m_blk = m // N_DEV

    def body(x_ref, w_hbm_ref, out_ref, xb_ref, xg_ref, wstage_ref,
             wb_ref, amax_ref, send_sems, recv_sems, send2_sems,
             recv2_sems, wdma_sems):
        my_i = lax.axis_index("i")

        xb_ref[:, :] = x_ref[:, :].astype(jnp.bfloat16)
        wdma = []
        for d in range(N_DEV):
            s = (my_i - d) % N_DEV
            cp = pltpu.make_async_copy(
                w_hbm_ref.at[pl.ds(s * k_per, k_per), :],
                wstage_ref.at[d % 4],
                wdma_sems.at[d],
            )
            if d < 4:
                cp.start()
            wdma.append(cp)

        if _ABLATE != "compute":
            barrier_sem = pltpu.get_barrier_semaphore()
            for d in range(1, N_DEV):
                pl.semaphore_signal(
                    barrier_sem, inc=1,
                    device_id=((my_i + d) % N_DEV,),
                    device_id_type=pl.DeviceIdType.MESH,
                )
            pl.semaphore_wait(barrier_sem, N_DEV - 1)

        a2a = []
        if _ABLATE != "compute":
            for d in range(1, N_DEV):
                t = (my_i + d) % N_DEV
                rdma = pltpu.make_async_remote_copy(
                    src_ref=xb_ref.at[pl.ds(t * m_blk, m_blk), :],
                    dst_ref=xg_ref.at[d],
                    send_sem=send_sems.at[d],
                    recv_sem=recv_sems.at[d],
                    device_id=(t,),
                    device_id_type=pl.DeviceIdType.MESH,
                )
                rdma.start()
                a2a.append(rdma)

        wdma[0].wait()
        wb_ref[0] = wstage_ref[0].astype(jnp.bfloat16)
        wdma[4].start()
        out_ref[:, :] = jnp.dot(
            xb_ref[pl.ds(my_i * m_blk, m_blk), :],
            wb_ref[0],
            preferred_element_type=jnp.float32,
        )
        for d in range(1, N_DEV):
            wdma[d].wait()
            wb_ref[d] = wstage_ref[d % 4].astype(jnp.bfloat16)
            if d + 4 < N_DEV:
                wdma[d + 4].start()

        for d in range(1, N_DEV):
            if _ABLATE != "compute":
                a2a[d - 1].wait()
            if _ABLATE != "comm":
                out_ref[:, :] = out_ref[:, :] + jnp.dot(
                    xg_ref[d],
                    wb_ref[d],
                    preferred_element_type=jnp.float32,
                )

        local_amax = jnp.max(jnp.abs(out_ref[:, :]))
        amax_ref[0, :] = jnp.full((128,), local_amax, jnp.float32)
        if _ABLATE != "compute" and not _NOAMAX:
            ax = []
            for d in range(1, N_DEV):
                t = (my_i + d) % N_DEV
                rdma = pltpu.make_async_remote_copy(
                    src_ref=amax_ref.at[0],
                    dst_ref=amax_ref.at[d],
                    send_sem=send2_sems.at[d],
                    recv_sem=recv2_sems.at[d],
                    device_id=(t,),
                    device_id_type=pl.DeviceIdType.MESH,
                )
                rdma.start()
                ax.append(rdma)
            for rdma in ax:
                rdma.wait()

        gmax = jnp.max(amax_ref[:, :])
        scale = gmax / 127.0
        q = jnp.clip(jnp.round(out_ref[:, :] / scale), -127.0, 127.0)
        out_ref[:, :] = q * scale

    return pl.pallas_call(
        body,
        out_shape=jax.ShapeDtypeStruct((m_blk, n), jnp.float32),
        in_specs=[
            pl.BlockSpec(memory_space=pltpu.VMEM),
            pl.BlockSpec(memory_space=pltpu.MemorySpace.HBM),
        ],
        out_specs=pl.BlockSpec(memory_space=pltpu.VMEM),
        scratch_shapes=[
            pltpu.VMEM((m, k_per), jnp.bfloat16),
            pltpu.VMEM((N_DEV, m_blk, k_per), jnp.bfloat16),
            pltpu.VMEM((4, k_per, n), jnp.float32),
            pltpu.VMEM((N_DEV, k_per, n), jnp.bfloat16),
            pltpu.VMEM((N_DEV, 128), jnp.float32),
            pltpu.SemaphoreType.DMA((N_DEV,)),
            pltpu.SemaphoreType.DMA((N_DEV,)),
            pltpu.SemaphoreType.DMA((N_DEV,)),
            pltpu.SemaphoreType.DMA((N_DEV,)),
            pltpu.SemaphoreType.DMA((N_DEV,)),
        ],
        compiler_params=pltpu.CompilerParams(
            collective_id=None if _ABLATE == "compute" else 0,
            vmem_limit_bytes=100 * 1024 * 1024,
        ),
    )(x, w_mat)
